# baseline (speedup 1.0000x reference)
import functools

import jax
import jax.numpy as jnp
from jax import lax
from jax.experimental import pallas as pl
from jax.experimental.pallas import tpu as pltpu
from jax.experimental.pallas import tpu_sc as plsc

_VOCAB = 100000
_DIM = 128
_EPS = 1e-12

_NORM_BLOCK = 2000

_NC = 2
_NS = 16
_NW = _NC * _NS
_G = 128
_NBUF = 4


def _norm_body(t_ref, w_ref, o_ref):
    x = t_ref[...]
    ms = jnp.mean(x * x, axis=-1, keepdims=True)
    y = x * lax.rsqrt(ms + _EPS) * w_ref[...]
    chunks = []
    for c in range(4):
        a = y[:, c * 32:c * 32 + 16].astype(jnp.bfloat16)
        b = y[:, c * 32 + 16:c * 32 + 32].astype(jnp.bfloat16)
        au = lax.bitcast_convert_type(a, jnp.uint16).astype(jnp.uint32)
        bu = lax.bitcast_convert_type(b, jnp.uint16).astype(jnp.uint32)
        chunks.append(au | (bu << 16))
    packed = jnp.concatenate(chunks, axis=-1)
    o_ref[...] = lax.bitcast_convert_type(packed, jnp.int32)


def _normalize_table(table, norm_weight):
    return pl.pallas_call(
        _norm_body,
        grid=(_VOCAB // _NORM_BLOCK,),
        in_specs=[
            pl.BlockSpec((_NORM_BLOCK, _DIM), lambda i: (i, 0)),
            pl.BlockSpec((1, _DIM), lambda i: (0, 0)),
        ],
        out_specs=pl.BlockSpec((_NORM_BLOCK, _DIM // 2), lambda i: (i, 0)),
        out_shape=jax.ShapeDtypeStruct((_VOCAB, _DIM // 2), jnp.int32),
    )(table, norm_weight.reshape(1, _DIM))


def _make_gather(n_ids):
    assert n_ids % (_NW * _G * _NBUF) == 0
    b_per_w = n_ids // _NW
    n_groups = b_per_w // _G
    n_outer = n_groups // _NBUF
    mesh = plsc.VectorSubcoreMesh(
        core_axis_name="c", subcore_axis_name="s",
        num_cores=_NC, num_subcores=_NS,
    )

    @functools.partial(
        pl.kernel,
        out_type=jax.ShapeDtypeStruct((n_ids * _DIM,), jnp.int32),
        mesh=mesh,
        scratch_types=[
            pltpu.VMEM((b_per_w,), jnp.int32),
            pltpu.VMEM((_NBUF, _G, _DIM // 2), jnp.int32),
            pltpu.VMEM((_NBUF, _G * _DIM), jnp.int32),
            pltpu.SemaphoreType.DMA((_NBUF,)),
            pltpu.SemaphoreType.DMA((_NBUF,)),
        ],
        compiler_params=pltpu.CompilerParams(use_tc_tiling_on_sc=False),
    )
    def gather_kernel(tab_hbm, ids_hbm, out_hbm, idx_v, gbuf, wbuf, gsem,
                      wsem):
        wid = lax.axis_index("s") * _NC + lax.axis_index("c")
        base = wid * b_per_w
        pltpu.sync_copy(ids_hbm.at[pl.ds(base, b_per_w)], idx_v)

        def start_gather(b, g):
            pltpu.async_copy(
                tab_hbm.at[idx_v.at[pl.ds(g * _G, _G)]],
                gbuf.at[b], gsem.at[b],
            )

        def wait_gather(b, g):
            pltpu.make_async_copy(
                tab_hbm.at[idx_v.at[pl.ds(g * _G, _G)]],
                gbuf.at[b], gsem.at[b],
            ).wait()

        def start_write(b, g):
            pltpu.async_copy(
                wbuf.at[b],
                out_hbm.at[pl.ds((base + g * _G) * _DIM, _G * _DIM)],
                wsem.at[b],
            )

        def wait_write(b, g):
            pltpu.make_async_copy(
                wbuf.at[b],
                out_hbm.at[pl.ds((base + g * _G) * _DIM, _G * _DIM)],
                wsem.at[b],
            ).wait()

        def widen(b):
            @plsc.parallel_loop(0, _G, unroll=8)
            def row(r):
                wq = pl.multiple_of(r * _DIM, _DIM)
                for c in range(4):
                    i = gbuf[b, r, pl.ds(c * 16, 16)]
                    wbuf[b, pl.ds(wq + c * 32, 16)] = i << 16
                    wbuf[b, pl.ds(wq + c * 32 + 16, 16)] = i & jnp.int32(-65536)

        def step(b, g, first, last):
            wait_gather(b, g)
            if not first:
                wait_write(b, g - _NBUF)
            widen(b)
            if not last:
                start_gather(b, g + _NBUF)
            start_write(b, g)

        for b in range(_NBUF):
            start_gather(b, b)
        for b in range(_NBUF):
            step(b, b, first=True, last=False)

        def outer(it, carry):
            for b in range(_NBUF):
                step(b, it * _NBUF + b, first=False, last=False)
            return carry

        lax.fori_loop(1, n_outer - 1, outer, 0)

        for b in range(_NBUF):
            step(b, (n_outer - 1) * _NBUF + b, first=False, last=True)
        for b in range(_NBUF):
            wait_write(b, n_groups - _NBUF + b)

    return gather_kernel


def kernel(input_ids, table, norm_weight):
    b, l = input_ids.shape
    normed = _normalize_table(table, norm_weight)
    ids_flat = input_ids.reshape(-1)
    out = _make_gather(ids_flat.size)(normed, ids_flat)
    return lax.bitcast_convert_type(out, jnp.float32).reshape(b, l, _DIM)


# speedup vs baseline: 1.6880x; 1.6880x over previous
import functools

import jax
import jax.numpy as jnp
from jax import lax
from jax.experimental import pallas as pl
from jax.experimental.pallas import tpu as pltpu
from jax.experimental.pallas import tpu_sc as plsc

_VOCAB = 100000
_DIM = 128
_EPS = 1e-12

_NORM_BLOCK = 2000

_NC = 2
_NS = 16
_NW = _NC * _NS
_G = 128


def _norm_body(t_ref, w_ref, o_ref):
    x = t_ref[...]
    ms = jnp.mean(x * x, axis=-1, keepdims=True)
    o_ref[...] = x * lax.rsqrt(ms + _EPS) * w_ref[...]


def _normalize_table(table, norm_weight):
    return pl.pallas_call(
        _norm_body,
        grid=(_VOCAB // _NORM_BLOCK,),
        in_specs=[
            pl.BlockSpec((_NORM_BLOCK, _DIM), lambda i: (i, 0)),
            pl.BlockSpec((1, _DIM), lambda i: (0, 0)),
        ],
        out_specs=pl.BlockSpec((_NORM_BLOCK, _DIM), lambda i: (i, 0)),
        out_shape=jax.ShapeDtypeStruct((_VOCAB, _DIM), jnp.float32),
    )(table, norm_weight.reshape(1, _DIM))


_NBUF = 4


def _make_gather(n_ids):
    assert n_ids % (_NW * _G * _NBUF) == 0
    b_per_w = n_ids // _NW
    n_groups = b_per_w // _G
    mesh = plsc.VectorSubcoreMesh(
        core_axis_name="c", subcore_axis_name="s",
        num_cores=_NC, num_subcores=_NS,
    )

    @functools.partial(
        pl.kernel,
        out_type=jax.ShapeDtypeStruct((n_ids, _DIM), jnp.float32),
        mesh=mesh,
        scratch_types=[
            pltpu.VMEM((b_per_w,), jnp.int32),
            pltpu.VMEM((_NBUF, _G, _DIM), jnp.float32),
            pltpu.SemaphoreType.DMA((_NBUF,)),
            pltpu.SemaphoreType.DMA((_NBUF,)),
        ],
    )
    def gather_kernel(tab_hbm, ids_hbm, out_hbm, idx_v, rows_v, gsem, wsem):
        wid = lax.axis_index("s") * _NC + lax.axis_index("c")
        base = wid * b_per_w
        pltpu.sync_copy(ids_hbm.at[pl.ds(base, b_per_w)], idx_v)

        def start_gather(b, g):
            pltpu.async_copy(
                tab_hbm.at[idx_v.at[pl.ds(g * _G, _G)]],
                rows_v.at[b], gsem.at[b],
            )

        def wait_gather(b, g):
            pltpu.make_async_copy(
                tab_hbm.at[idx_v.at[pl.ds(g * _G, _G)]],
                rows_v.at[b], gsem.at[b],
            ).wait()

        def start_write(b, g):
            pltpu.async_copy(
                rows_v.at[b], out_hbm.at[pl.ds(base + g * _G, _G)], wsem.at[b]
            )

        def wait_write(b, g):
            pltpu.make_async_copy(
                rows_v.at[b], out_hbm.at[pl.ds(base + g * _G, _G)], wsem.at[b]
            ).wait()

        for b in range(_NBUF):
            start_gather(b, b)

        def outer(it, carry):
            g0 = it * _NBUF
            for b in range(_NBUF):
                g = g0 + b
                wait_gather(b, g)
                start_write(b, g)
                wait_write(b, g)
                start_gather(b, g + _NBUF)
            return carry

        lax.fori_loop(0, n_groups // _NBUF - 1, outer, 0)

        for b in range(_NBUF):
            g = n_groups - _NBUF + b
            wait_gather(b, g)
            start_write(b, g)
        for b in range(_NBUF):
            g = n_groups - _NBUF + b
            wait_write(b, g)

    return gather_kernel


def kernel(input_ids, table, norm_weight):
    b, l = input_ids.shape
    normed = _normalize_table(table, norm_weight)
    ids_flat = input_ids.reshape(-1)
    out = _make_gather(ids_flat.size)(normed, ids_flat)
    return out.reshape(b, l, _DIM)


# speedup vs baseline: 1.6886x; 1.0004x over previous
import functools

import jax
import jax.numpy as jnp
from jax import lax
from jax.experimental import pallas as pl
from jax.experimental.pallas import tpu as pltpu
from jax.experimental.pallas import tpu_sc as plsc

_VOCAB = 100000
_DIM = 128
_EPS = 1e-12

_NORM_BLOCK = 2000

_NC = 2
_NS = 16
_NW = _NC * _NS
_G = 128


def _norm_body(t_ref, w_ref, o_ref):
    x = t_ref[...]
    ms = jnp.mean(x * x, axis=-1, keepdims=True)
    o_ref[...] = x * lax.rsqrt(ms + _EPS) * w_ref[...]


def _normalize_table(table, norm_weight):
    return pl.pallas_call(
        _norm_body,
        grid=(_VOCAB // _NORM_BLOCK,),
        in_specs=[
            pl.BlockSpec((_NORM_BLOCK, _DIM), lambda i: (i, 0)),
            pl.BlockSpec((1, _DIM), lambda i: (0, 0)),
        ],
        out_specs=pl.BlockSpec((_NORM_BLOCK, _DIM), lambda i: (i, 0)),
        out_shape=jax.ShapeDtypeStruct((_VOCAB, _DIM), jnp.float32),
    )(table, norm_weight.reshape(1, _DIM))


_NBUF = 5


def _make_gather(n_ids):
    assert n_ids % (_NW * _G * _NBUF) == 0
    b_per_w = n_ids // _NW
    n_groups = b_per_w // _G
    mesh = plsc.VectorSubcoreMesh(
        core_axis_name="c", subcore_axis_name="s",
        num_cores=_NC, num_subcores=_NS,
    )

    @functools.partial(
        pl.kernel,
        out_type=jax.ShapeDtypeStruct((n_ids, _DIM), jnp.float32),
        mesh=mesh,
        scratch_types=[
            pltpu.VMEM((b_per_w,), jnp.int32),
            pltpu.VMEM((_NBUF, _G, _DIM), jnp.float32),
            pltpu.SemaphoreType.DMA((_NBUF,)),
            pltpu.SemaphoreType.DMA((_NBUF,)),
        ],
    )
    def gather_kernel(tab_hbm, ids_hbm, out_hbm, idx_v, rows_v, gsem, wsem):
        wid = lax.axis_index("s") * _NC + lax.axis_index("c")
        base = wid * b_per_w
        pltpu.sync_copy(ids_hbm.at[pl.ds(base, b_per_w)], idx_v)

        def start_gather(b, g):
            pltpu.async_copy(
                tab_hbm.at[idx_v.at[pl.ds(g * _G, _G)]],
                rows_v.at[b], gsem.at[b],
            )

        def wait_gather(b, g):
            pltpu.make_async_copy(
                tab_hbm.at[idx_v.at[pl.ds(g * _G, _G)]],
                rows_v.at[b], gsem.at[b],
            ).wait()

        def start_write(b, g):
            pltpu.async_copy(
                rows_v.at[b], out_hbm.at[pl.ds(base + g * _G, _G)], wsem.at[b]
            )

        def wait_write(b, g):
            pltpu.make_async_copy(
                rows_v.at[b], out_hbm.at[pl.ds(base + g * _G, _G)], wsem.at[b]
            ).wait()

        for b in range(_NBUF):
            start_gather(b, b)

        def outer(it, carry):
            g0 = it * _NBUF
            for b in range(_NBUF):
                g = g0 + b
                wait_gather(b, g)
                start_write(b, g)
                wait_write(b, g)
                start_gather(b, g + _NBUF)
            return carry

        lax.fori_loop(0, n_groups // _NBUF - 1, outer, 0)

        for b in range(_NBUF):
            g = n_groups - _NBUF + b
            wait_gather(b, g)
            start_write(b, g)
        for b in range(_NBUF):
            g = n_groups - _NBUF + b
            wait_write(b, g)

    return gather_kernel


def kernel(input_ids, table, norm_weight):
    b, l = input_ids.shape
    normed = _normalize_table(table, norm_weight)
    ids_flat = input_ids.reshape(-1)
    out = _make_gather(ids_flat.size)(normed, ids_flat)
    return out.reshape(b, l, _DIM)
